# Initial kernel scaffold; baseline (speedup 1.0000x reference)
import jax
import jax.numpy as jnp
from jax.experimental import pallas as pl


def kernel(x, edge_index1, e_id1, e_weight1, edge_index2, e_id2, e_weight2, xnode, params):
    raise NotImplementedError("write your pallas kernel here")


# speedup vs baseline: 12.9524x; 12.9524x over previous
import functools

import jax
import jax.numpy as jnp
from jax import lax
from jax.experimental import pallas as pl
from jax.experimental.pallas import tpu as pltpu
from jax.experimental.pallas import tpu_sc as plsc

N1 = 32768
N2 = 8192
D_IN = 128
D_HID = 64
D1 = 80
D2 = 16
NC = 2
NS = 16
SUB = 128


def _make_sc_seg(E, D):
  EW = E // (NC * NS)
  CHUNK = min(4096, EW)
  CBUF = EW + SUB
  RPT = N2 // NS
  mesh = plsc.VectorSubcoreMesh(core_axis_name="c", subcore_axis_name="s")

  @functools.partial(
      pl.kernel,
      out_type=jax.ShapeDtypeStruct((NC, N2, D), jnp.float32),
      mesh=mesh,
      compiler_params=pltpu.CompilerParams(
          use_tc_tiling_on_sc=False, needs_layout_passes=False),
      scratch_types=[
          pltpu.VMEM((CHUNK,), jnp.int32),
          pltpu.VMEM((CHUNK,), jnp.int32),
          pltpu.VMEM((CHUNK,), jnp.float32),
          pltpu.VMEM((CBUF,), jnp.int32),
          pltpu.VMEM((CBUF,), jnp.int32),
          pltpu.VMEM((CBUF,), jnp.float32),
          pltpu.VMEM((SUB,), jnp.int32),
          pltpu.VMEM((SUB,), jnp.int32),
          pltpu.VMEM((SUB, D), jnp.float32),
          pltpu.VMEM_SHARED((N2, D), jnp.float32),
          pltpu.SemaphoreType.DMA,
      ],
  )
  def sc_seg(src_hbm, dst_hbm, ew_hbm, tab_hbm, out_hbm,
             stage_s, stage_d, stage_w, csrc, cdst, cew,
             gidx, didx, rows, acc, sem):
    c = lax.axis_index("c")
    s = lax.axis_index("s")
    base = (c * NS + s) * EW

    zf = jnp.zeros((16,), jnp.float32)

    def zrow(i, _):
      for j in range(D // 16):
        rows[i, pl.ds(j * 16, 16)] = zf
      return 0

    lax.fori_loop(0, SUB, zrow, 0)
    for k in range(RPT // SUB):
      pltpu.sync_copy(rows, acc.at[pl.ds(s * RPT + k * SUB, SUB)])
    plsc.subcore_barrier()

    def chunk_body(ci, cnt):
      off = base + ci * CHUNK
      pltpu.async_copy(src_hbm.at[pl.ds(off, CHUNK)], stage_s, sem).wait()
      pltpu.async_copy(dst_hbm.at[pl.ds(off, CHUNK)], stage_d, sem).wait()
      pltpu.async_copy(ew_hbm.at[pl.ds(off, CHUNK)], stage_w, sem).wait()

      def grp(gi, cnt):
        lane = lax.iota(jnp.int32, 16)
        trash = CBUF - 16 + lane
        d = stage_d[pl.ds(gi * 16, 16)]
        sv = stage_s[pl.ds(gi * 16, 16)]
        wv = stage_w[pl.ds(gi * 16, 16)]
        m = d < N2
        cum = plsc.cumsum(jnp.where(m, 1, 0))
        pos = jnp.where(m, jnp.full((16,), cnt, jnp.int32) + cum - 1, trash)
        plsc.store_scatter(cdst, [pos], d)
        plsc.store_scatter(csrc, [pos], sv)
        plsc.store_scatter(cew, [pos], wv)
        return cnt + jnp.max(cum)

      return lax.fori_loop(0, CHUNK // 16, grp, cnt)

    cnt = lax.fori_loop(0, EW // CHUNK, chunk_body, jnp.int32(0))

    zi = jnp.zeros((16,), jnp.int32)
    for k in range(SUB // 16):
      cdst[pl.ds(cnt + k * 16, 16)] = zi
      csrc[pl.ds(cnt + k * 16, 16)] = zi
      cew[pl.ds(cnt + k * 16, 16)] = zf

    n_sub = (cnt + SUB - 1) // SUB

    def sub_body(k, _):
      kb = k * SUB
      for t in range(SUB // 16):
        gidx[pl.ds(t * 16, 16)] = csrc[pl.ds(kb + t * 16, 16)]
        didx[pl.ds(t * 16, 16)] = cdst[pl.ds(kb + t * 16, 16)]
      pltpu.async_copy(tab_hbm.at[gidx], rows, sem).wait()

      def sgrp(g, _):
        for i2 in range(16):
          il = g * 16 + i2
          wsp = plsc.load_gather(cew, [jnp.full((16,), kb + il, jnp.int32)])
          for j in range(D // 16):
            rows[il, pl.ds(j * 16, 16)] = rows[il, pl.ds(j * 16, 16)] * wsp
        return 0

      lax.fori_loop(0, SUB // 16, sgrp, 0)
      pltpu.sync_copy(rows, acc.at[didx], add=True)
      return 0

    lax.fori_loop(0, n_sub, sub_body, 0)
    plsc.subcore_barrier()

    for k in range(RPT // SUB):
      off = s * RPT + k * SUB
      pltpu.sync_copy(acc.at[pl.ds(off, SUB)], rows)
      pltpu.sync_copy(rows, out_hbm.at[c, pl.ds(off, SUB)])

  return sc_seg


_sc_seg1 = _make_sc_seg(524288, D1)
_sc_seg2 = _make_sc_seg(131072, D2)


def _tc1_body(x_ref, w_ref, o_ref):
  xp = jnp.dot(x_ref[...], w_ref[...], preferred_element_type=jnp.float32)
  b = xp.shape[0]
  o_ref[...] = jnp.concatenate(
      [xp, jnp.ones((b, 1), jnp.float32), jnp.zeros((b, D1 - D_HID - 1), jnp.float32)],
      axis=1)


def _tc2_body(parts_ref, x8_ref, w1rT_ref, b1_ref, w2lT_ref, w2rT_ref, b2_ref,
              hp_ref, hr_ref):
  num = parts_ref[0] + parts_ref[1]
  den = num[:, D_HID:D_HID + 1]
  agg = num[:, :D_HID] / (den + 1e-16)
  h8 = jax.nn.relu(
      agg + jnp.dot(x8_ref[...], w1rT_ref[...], preferred_element_type=jnp.float32)
      + b1_ref[...])
  hp = jnp.dot(h8, w2lT_ref[...], preferred_element_type=jnp.float32)
  hr = jnp.dot(h8, w2rT_ref[...], preferred_element_type=jnp.float32) + b2_ref[...]
  b = h8.shape[0]
  hp_ref[...] = jnp.concatenate(
      [hp, jnp.ones((b, 1), jnp.float32), jnp.zeros((b, D2 - 2), jnp.float32)],
      axis=1)
  hr_ref[...] = hr


def _tc3_body(parts_ref, hr_ref, xnode_ref,
              wattT_ref, batt_ref, attbn_s_ref, attbn_b_ref,
              a0T_ref, a0b_ref, bn0_s_ref, bn0_b_ref,
              a1T_ref, a1b_ref, bn1_s_ref, bn1_b_ref,
              a2T_ref, a2b_ref, bn2_s_ref, bn2_b_ref,
              a3T_ref, a3b_ref, bn3_s_ref, bn3_b_ref,
              a4T_ref, a4b_ref, bn4_s_ref, bn4_b_ref,
              l2T_ref, l2b_ref, bn5_s_ref, bn5_b_ref,
              lT_ref, lb_ref, o_ref):
  relu = jax.nn.relu

  def mm(a, b):
    return jnp.dot(a, b, preferred_element_type=jnp.float32)

  num = parts_ref[0] + parts_ref[1]
  gcnx = num[:, 0:1] / (num[:, 1:2] + 1e-16) + hr_ref[...]
  xin = jnp.concatenate([xnode_ref[...], gcnx], axis=1)
  z = mm(xin, wattT_ref[...]) + batt_ref[...]
  prob = jax.nn.softmax(z, axis=1)
  xin = xin * prob + xin
  xin = xin * attbn_s_ref[...] + attbn_b_ref[...]
  h0 = relu(mm(xin, a0T_ref[...]) + a0b_ref[...]) * bn0_s_ref[...] + bn0_b_ref[...]
  h1 = relu(mm(h0, a1T_ref[...]) + a1b_ref[...]) * bn1_s_ref[...] + bn1_b_ref[...]
  h2 = relu(mm(h1, a2T_ref[...]) + a2b_ref[...]) * bn2_s_ref[...] + bn2_b_ref[...]
  h3 = relu(mm(h2, a3T_ref[...]) + a3b_ref[...]) * bn3_s_ref[...] + bn3_b_ref[...] + h1
  h4 = relu(mm(h3, a4T_ref[...]) + a4b_ref[...]) * bn4_s_ref[...] + bn4_b_ref[...] + h0
  h5 = relu(mm(h4, l2T_ref[...]) + l2b_ref[...]) * bn5_s_ref[...] + bn5_b_ref[...] + xin
  o = mm(h5, lT_ref[...]) + lb_ref[...]
  pm25 = o[:, 0:1]
  pm10 = o[:, 1:2]
  o_ref[...] = jnp.concatenate([pm25, pm10, relu(pm25 - pm10)], axis=1)


def _row2(a):
  return a.reshape(1, -1)


def kernel(x, edge_index1, e_id1, e_weight1, edge_index2, e_id2, e_weight2,
           xnode, params):
  del e_id1, e_id2
  p = params
  inv = 1.0 / jnp.sqrt(jnp.float32(1.0 + 1e-5))

  B1 = 2048
  xp_pad = pl.pallas_call(
      _tc1_body,
      grid=(N1 // B1,),
      in_specs=[
          pl.BlockSpec((B1, D_IN), lambda i: (i, 0)),
          pl.BlockSpec((D_IN, D_HID), lambda i: (0, 0)),
      ],
      out_specs=pl.BlockSpec((B1, D1), lambda i: (i, 0)),
      out_shape=jax.ShapeDtypeStruct((N1, D1), jnp.float32),
  )(x[:N1], p['W1_l'].T)

  parts1 = _sc_seg1(edge_index1[0], edge_index1[1], e_weight1, xp_pad)

  B2 = 1024
  hp_pad, hr = pl.pallas_call(
      _tc2_body,
      grid=(N2 // B2,),
      in_specs=[
          pl.BlockSpec((NC, B2, D1), lambda i: (0, i, 0)),
          pl.BlockSpec((B2, D_IN), lambda i: (i, 0)),
          pl.BlockSpec((D_IN, D_HID), lambda i: (0, 0)),
          pl.BlockSpec((1, D_HID), lambda i: (0, 0)),
          pl.BlockSpec((D_HID, 1), lambda i: (0, 0)),
          pl.BlockSpec((D_HID, 1), lambda i: (0, 0)),
          pl.BlockSpec((1, 1), lambda i: (0, 0)),
      ],
      out_specs=[
          pl.BlockSpec((B2, D2), lambda i: (i, 0)),
          pl.BlockSpec((B2, 1), lambda i: (i, 0)),
      ],
      out_shape=[
          jax.ShapeDtypeStruct((N2, D2), jnp.float32),
          jax.ShapeDtypeStruct((N2, 1), jnp.float32),
      ],
  )(parts1, x[:N2], p['W1_r'].T, _row2(p['b1']), p['W2_l'].T, p['W2_r'].T,
    _row2(p['b2']))

  parts2 = _sc_seg2(edge_index2[0], edge_index2[1], e_weight2, hp_pad)

  B3 = 1024
  w_arrays = [
      p['Watt0'].T, _row2(p['batt0']),
      _row2(p['attbn0_g'] * inv), _row2(p['attbn0_b']),
  ]
  for i in range(5):
    w_arrays.append(p['A%d_W' % i].T)
    w_arrays.append(_row2(p['A%d_b' % i]))
    w_arrays.append(_row2(p['bn%d_g' % i] * inv))
    w_arrays.append(_row2(p['bn%d_b' % i]))
  w_arrays += [p['L2_W'].T, _row2(p['L2_b']), _row2(p['bn5_g'] * inv),
               _row2(p['bn5_b']), p['L_W'].T, _row2(p['L_b'])]
  w_specs = [pl.BlockSpec(a.shape, lambda i: (0, 0)) for a in w_arrays]

  out = pl.pallas_call(
      _tc3_body,
      grid=(N2 // B3,),
      in_specs=[
          pl.BlockSpec((NC, B3, D2), lambda i: (0, i, 0)),
          pl.BlockSpec((B3, 1), lambda i: (i, 0)),
          pl.BlockSpec((B3, D_IN), lambda i: (i, 0)),
      ] + w_specs,
      out_specs=pl.BlockSpec((B3, 3), lambda i: (i, 0)),
      out_shape=jax.ShapeDtypeStruct((N2, 3), jnp.float32),
  )(parts2, hr, xnode, *w_arrays)
  return out
